# baseline (speedup 1.0000x reference)
import functools

import jax
import jax.numpy as jnp
from jax import lax
from jax.experimental import pallas as pl
from jax.experimental.pallas import tpu as pltpu
from jax.experimental.pallas import tpu_sc as plsc

D = 128
L = 16
NC = 2
NS = 16
NW = NC * NS
GB = 128
EPS = 1e-5


def _rsqrt(x):
    xi = plsc.bitcast(x, jnp.int32)
    yi = jnp.int32(0x5F3759DF) - lax.shift_right_logical(xi, 1)
    y = plsc.bitcast(yi, jnp.float32)
    half_x = 0.5 * x
    for _ in range(2):
        y = y * (1.5 - half_x * y * y)
    return y


def _make_sc_kernel(num_tokens):
    assert num_tokens % (NW * GB) == 0
    ch = num_tokens // (NW * GB)
    assert ch % 2 == 0

    mesh = plsc.VectorSubcoreMesh(
        core_axis_name="c", subcore_axis_name="s", num_cores=NC,
        num_subcores=NS)

    @functools.partial(
        pl.kernel,
        mesh=mesh,
        out_type=jax.ShapeDtypeStruct((num_tokens, D), jnp.float32),
        compiler_params=pltpu.CompilerParams(needs_layout_passes=False),
        scratch_types=[
            pltpu.VMEM((ch, GB), jnp.int32),
            pltpu.VMEM((2, GB, D), jnp.float32),
            pltpu.VMEM((2, GB, D), jnp.float32),
            pltpu.VMEM((D,), jnp.float32),
            pltpu.VMEM((D,), jnp.float32),
            pltpu.VMEM((GB // L, L, 17), jnp.float32),
            pltpu.VMEM((GB // L, L, 17), jnp.float32),
            pltpu.SemaphoreType.DMA,
            pltpu.SemaphoreType.DMA,
            pltpu.SemaphoreType.DMA,
            pltpu.SemaphoreType.DMA,
        ],
    )
    def k(table_hbm, idx_hbm, gamma_hbm, beta_hbm, out_hbm,
          idx_v, in_v, out_v, gamma_v, beta_v, pbuf, qbuf,
          in_sem0, in_sem1, out_sem0, out_sem1):
        wid = lax.axis_index("s") * NC + lax.axis_index("c")
        tok0 = wid * ch * GB

        pltpu.sync_copy(idx_hbm.at[wid], idx_v)
        pltpu.sync_copy(gamma_hbm, gamma_v)
        pltpu.sync_copy(beta_hbm, beta_v)

        in_sems = (in_sem0, in_sem1)
        out_sems = (out_sem0, out_sem1)

        def start_in(c, b):
            pltpu.make_async_copy(
                table_hbm.at[idx_v.at[c]], in_v.at[b], in_sems[b]).start()

        def wait_in(b):
            pltpu.make_async_copy(
                table_hbm.at[idx_v.at[0]], in_v.at[b], in_sems[b]).wait()

        def start_out(c, b):
            pltpu.make_async_copy(
                out_v.at[b], out_hbm.at[pl.ds(tok0 + c * GB, GB)],
                out_sems[b]).start()

        def wait_out(b):
            pltpu.make_async_copy(
                out_v.at[b], out_hbm.at[pl.ds(tok0, GB)], out_sems[b]).wait()

        gvs = [gamma_v[pl.ds(j * L, L)] for j in range(D // L)]
        bvs = [beta_v[pl.ds(j * L, L)] for j in range(D // L)]

        def compute(b):
            inb = in_v.at[b]
            outb = out_v.at[b]

            @plsc.parallel_loop(0, GB // L, 1, unroll=4)
            def group_body(g):
                pb = pbuf.at[g]
                qb = qbuf.at[g]
                for kk in range(L):
                    r = g * L + kk
                    vs = [inb[r, pl.ds(j * L, L)] for j in range(D // L)]
                    s = ((vs[0] + vs[1]) + (vs[2] + vs[3])) + (
                        (vs[4] + vs[5]) + (vs[6] + vs[7]))
                    sqs = [v * v for v in vs]
                    q = ((sqs[0] + sqs[1]) + (sqs[2] + sqs[3])) + (
                        (sqs[4] + sqs[5]) + (sqs[6] + sqs[7]))
                    pb[kk, pl.ds(0, L)] = s
                    qb[kk, pl.ds(0, L)] = q

                lanes = lax.iota(jnp.int32, L)
                ps = [plsc.load_gather(pb, [lanes, jnp.full((L,), j, jnp.int32)])
                      for j in range(L)]
                qs = [plsc.load_gather(qb, [lanes, jnp.full((L,), j, jnp.int32)])
                      for j in range(L)]

                def tree(xs):
                    while len(xs) > 1:
                        xs = [a + b for a, b in zip(xs[::2], xs[1::2])]
                    return xs[0]

                s = tree(ps)
                q = tree(qs)
                mean = s * (1.0 / D)
                var = q * (1.0 / D) - mean * mean
                rstd = _rsqrt(var + EPS)

                for kk in range(L):
                    m = jnp.broadcast_to(mean[kk], (L,))
                    sd = jnp.broadcast_to(rstd[kk], (L,))
                    r = g * L + kk
                    for j in range(D // L):
                        v = inb[r, pl.ds(j * L, L)]
                        outb[r, pl.ds(j * L, L)] = (
                            (v - m) * sd * gvs[j] + bvs[j])

        start_in(0, 0)

        def pair_body(p, _):
            c0 = 2 * p
            start_in(c0 + 1, 1)

            pl.when(p > 0)(lambda: wait_out(0))
            wait_in(0)
            compute(0)
            start_out(c0, 0)

            pl.when(p < ch // 2 - 1)(lambda: start_in(c0 + 2, 0))

            pl.when(p > 0)(lambda: wait_out(1))
            wait_in(1)
            compute(1)
            start_out(c0 + 1, 1)
            return 0

        lax.fori_loop(0, ch // 2, pair_body, 0)
        wait_out(0)
        wait_out(1)

    return k


_sc_kernel = _make_sc_kernel(1024 * 200)


@jax.jit
def kernel(x, table, gamma, beta):
    b, s = x.shape
    idx = x.reshape(NW, b * s // (NW * GB), GB).astype(jnp.int32)
    out = _sc_kernel(table, idx, gamma, beta)
    return out.reshape(b, s, D)


# speedup vs baseline: 2.8812x; 2.8812x over previous
import functools

import jax
import jax.numpy as jnp
from jax import lax
from jax.experimental import pallas as pl
from jax.experimental.pallas import tpu as pltpu
from jax.experimental.pallas import tpu_sc as plsc

D = 128
L = 16
NC = 2
NS = 16
NW = NC * NS
GB = 128
EPS = 1e-5


def _rsqrt(x):
    xi = plsc.bitcast(x, jnp.int32)
    yi = jnp.int32(0x5F3759DF) - lax.shift_right_logical(xi, 1)
    y = plsc.bitcast(yi, jnp.float32)
    half_x = 0.5 * x
    for _ in range(2):
        y = y * (1.5 - half_x * y * y)
    return y


def _make_sc_kernel(num_tokens):
    assert num_tokens % (NW * GB) == 0
    ch = num_tokens // (NW * GB)
    assert ch % 2 == 0

    mesh = plsc.VectorSubcoreMesh(
        core_axis_name="c", subcore_axis_name="s", num_cores=NC,
        num_subcores=NS)

    @functools.partial(
        pl.kernel,
        mesh=mesh,
        out_type=jax.ShapeDtypeStruct((num_tokens, D), jnp.float32),
        compiler_params=pltpu.CompilerParams(needs_layout_passes=False),
        scratch_types=[
            pltpu.VMEM((ch, GB), jnp.int32),
            pltpu.VMEM((2, GB, D), jnp.float32),
            pltpu.VMEM((2, GB, D), jnp.float32),
            pltpu.VMEM((D,), jnp.float32),
            pltpu.VMEM((D,), jnp.float32),
            pltpu.VMEM((GB // L, L, 17), jnp.float32),
            pltpu.VMEM((GB // L, L, 17), jnp.float32),
            pltpu.SemaphoreType.DMA,
            pltpu.SemaphoreType.DMA,
            pltpu.SemaphoreType.DMA,
            pltpu.SemaphoreType.DMA,
        ],
    )
    def k(table_hbm, idx_hbm, gamma_hbm, beta_hbm, out_hbm,
          idx_v, in_v, out_v, gamma_v, beta_v, pbuf, qbuf,
          in_sem0, in_sem1, out_sem0, out_sem1):
        wid = lax.axis_index("s") * NC + lax.axis_index("c")
        tok0 = wid * ch * GB

        pltpu.sync_copy(idx_hbm.at[wid], idx_v)
        pltpu.sync_copy(gamma_hbm, gamma_v)
        pltpu.sync_copy(beta_hbm, beta_v)

        in_sems = (in_sem0, in_sem1)
        out_sems = (out_sem0, out_sem1)

        def start_in(c, b):
            pltpu.make_async_copy(
                table_hbm.at[idx_v.at[c]], in_v.at[b], in_sems[b]).start()

        def wait_in(b):
            pltpu.make_async_copy(
                table_hbm.at[idx_v.at[0]], in_v.at[b], in_sems[b]).wait()

        def start_out(c, b):
            pltpu.make_async_copy(
                out_v.at[b], out_hbm.at[pl.ds(tok0 + c * GB, GB)],
                out_sems[b]).start()

        def wait_out(b):
            pltpu.make_async_copy(
                out_v.at[b], out_hbm.at[pl.ds(tok0, GB)], out_sems[b]).wait()

        gvs = [gamma_v[pl.ds(j * L, L)] for j in range(D // L)]
        bvs = [beta_v[pl.ds(j * L, L)] for j in range(D // L)]

        def compute(b):
            inb = in_v.at[b]
            outb = out_v.at[b]

            @plsc.parallel_loop(0, GB // L, 1, unroll=2)
            def group_body(g):
                pb = pbuf.at[g]
                qb = qbuf.at[g]
                for kk in range(L):
                    r = g * L + kk
                    vs = [inb[r, pl.ds(j * L, L)] for j in range(D // L)]
                    s = ((vs[0] + vs[1]) + (vs[2] + vs[3])) + (
                        (vs[4] + vs[5]) + (vs[6] + vs[7]))
                    sqs = [v * v for v in vs]
                    q = ((sqs[0] + sqs[1]) + (sqs[2] + sqs[3])) + (
                        (sqs[4] + sqs[5]) + (sqs[6] + sqs[7]))
                    pb[kk, pl.ds(0, L)] = s
                    qb[kk, pl.ds(0, L)] = q

                lanes = lax.iota(jnp.int32, L)
                ps = [plsc.load_gather(pb, [lanes, jnp.full((L,), j, jnp.int32)])
                      for j in range(L)]
                qs = [plsc.load_gather(qb, [lanes, jnp.full((L,), j, jnp.int32)])
                      for j in range(L)]

                def tree(xs):
                    while len(xs) > 1:
                        xs = [a + b for a, b in zip(xs[::2], xs[1::2])]
                    return xs[0]

                s = tree(ps)
                q = tree(qs)
                mean = s * (1.0 / D)
                var = q * (1.0 / D) - mean * mean
                rstd = _rsqrt(var + EPS)

                for kk in range(L):
                    m = jnp.broadcast_to(mean[kk], (L,))
                    sd = jnp.broadcast_to(rstd[kk], (L,))
                    r = g * L + kk
                    for j in range(D // L):
                        v = inb[r, pl.ds(j * L, L)]
                        outb[r, pl.ds(j * L, L)] = (
                            (v - m) * sd * gvs[j] + bvs[j])

        start_in(0, 0)

        def pair_body(p, _):
            c0 = 2 * p
            start_in(c0 + 1, 1)

            pl.when(p > 0)(lambda: wait_out(0))
            wait_in(0)
            compute(0)
            start_out(c0, 0)

            pl.when(p < ch // 2 - 1)(lambda: start_in(c0 + 2, 0))

            pl.when(p > 0)(lambda: wait_out(1))
            wait_in(1)
            compute(1)
            start_out(c0 + 1, 1)
            return 0

        lax.fori_loop(0, ch // 2, pair_body, 0)
        wait_out(0)
        wait_out(1)

    return k


_sc_kernel = _make_sc_kernel(1024 * 200)


@jax.jit
def kernel(x, table, gamma, beta):
    b, s = x.shape
    idx = x.reshape(NW, b * s // (NW * GB), GB).astype(jnp.int32)
    out = _sc_kernel(table, idx, gamma, beta)
    return out.reshape(b, s, D)


# speedup vs baseline: 3.1864x; 1.1059x over previous
import functools

import jax
import jax.numpy as jnp
from jax import lax
from jax.experimental import pallas as pl
from jax.experimental.pallas import tpu as pltpu
from jax.experimental.pallas import tpu_sc as plsc

D = 128
L = 16
NC = 2
NS = 16
NW = NC * NS
GB = 128
EPS = 1e-5


def _rsqrt(x):
    xi = plsc.bitcast(x, jnp.int32)
    yi = jnp.int32(0x5F3759DF) - lax.shift_right_logical(xi, 1)
    y = plsc.bitcast(yi, jnp.float32)
    half_x = 0.5 * x
    for _ in range(2):
        y = y * (1.5 - half_x * y * y)
    return y


def _make_sc_kernel(num_tokens):
    assert num_tokens % (NW * GB) == 0
    ch = num_tokens // (NW * GB)
    assert ch % 2 == 0

    mesh = plsc.VectorSubcoreMesh(
        core_axis_name="c", subcore_axis_name="s", num_cores=NC,
        num_subcores=NS)

    @functools.partial(
        pl.kernel,
        mesh=mesh,
        out_type=jax.ShapeDtypeStruct((num_tokens, D), jnp.float32),
        compiler_params=pltpu.CompilerParams(needs_layout_passes=False),
        scratch_types=[
            pltpu.VMEM((ch, GB), jnp.int32),
            pltpu.VMEM((2, GB, D), jnp.float32),
            pltpu.VMEM((2, GB, D), jnp.float32),
            pltpu.VMEM((D,), jnp.float32),
            pltpu.VMEM((D,), jnp.float32),
            pltpu.VMEM((GB // L, L, 17), jnp.float32),
            pltpu.VMEM((GB // L, L, 17), jnp.float32),
            pltpu.SemaphoreType.DMA,
            pltpu.SemaphoreType.DMA,
            pltpu.SemaphoreType.DMA,
            pltpu.SemaphoreType.DMA,
        ],
    )
    def k(table_hbm, idx_hbm, gamma_hbm, beta_hbm, out_hbm,
          idx_v, in_v, out_v, gamma_v, beta_v, pbuf, qbuf,
          in_sem0, in_sem1, out_sem0, out_sem1):
        wid = lax.axis_index("s") * NC + lax.axis_index("c")
        tok0 = wid * ch * GB

        pltpu.sync_copy(idx_hbm.at[wid], idx_v)
        pltpu.sync_copy(gamma_hbm, gamma_v)
        pltpu.sync_copy(beta_hbm, beta_v)

        in_sems = (in_sem0, in_sem1)
        out_sems = (out_sem0, out_sem1)

        def start_in(c, b):
            pltpu.make_async_copy(
                table_hbm.at[idx_v.at[c]], in_v.at[b], in_sems[b]).start()

        def wait_in(b):
            pltpu.make_async_copy(
                table_hbm.at[idx_v.at[0]], in_v.at[b], in_sems[b]).wait()

        def start_out(c, b):
            pltpu.make_async_copy(
                out_v.at[b], out_hbm.at[pl.ds(tok0 + c * GB, GB)],
                out_sems[b]).start()

        def wait_out(b):
            pltpu.make_async_copy(
                out_v.at[b], out_hbm.at[pl.ds(tok0, GB)], out_sems[b]).wait()

        gvs = [gamma_v[pl.ds(j * L, L)] for j in range(D // L)]
        bvs = [beta_v[pl.ds(j * L, L)] for j in range(D // L)]

        def compute(b):
            inb = in_v.at[b]
            outb = out_v.at[b]

            @plsc.parallel_loop(0, GB // L, 1, unroll=1)
            def group_body(g):
                pb = pbuf.at[g]
                qb = qbuf.at[g]
                for kk in range(L):
                    r = g * L + kk
                    vs = [inb[r, pl.ds(j * L, L)] for j in range(D // L)]
                    s = ((vs[0] + vs[1]) + (vs[2] + vs[3])) + (
                        (vs[4] + vs[5]) + (vs[6] + vs[7]))
                    sqs = [v * v for v in vs]
                    q = ((sqs[0] + sqs[1]) + (sqs[2] + sqs[3])) + (
                        (sqs[4] + sqs[5]) + (sqs[6] + sqs[7]))
                    pb[kk, pl.ds(0, L)] = s
                    qb[kk, pl.ds(0, L)] = q

                lanes = lax.iota(jnp.int32, L)
                ps = [plsc.load_gather(pb, [lanes, jnp.full((L,), j, jnp.int32)])
                      for j in range(L)]
                qs = [plsc.load_gather(qb, [lanes, jnp.full((L,), j, jnp.int32)])
                      for j in range(L)]

                def tree(xs):
                    while len(xs) > 1:
                        xs = [a + b for a, b in zip(xs[::2], xs[1::2])]
                    return xs[0]

                s = tree(ps)
                q = tree(qs)
                mean = s * (1.0 / D)
                var = q * (1.0 / D) - mean * mean
                rstd = _rsqrt(var + EPS)

                for kk in range(L):
                    m = jnp.broadcast_to(mean[kk], (L,))
                    sd = jnp.broadcast_to(rstd[kk], (L,))
                    r = g * L + kk
                    for j in range(D // L):
                        v = inb[r, pl.ds(j * L, L)]
                        outb[r, pl.ds(j * L, L)] = (
                            (v - m) * sd * gvs[j] + bvs[j])

        start_in(0, 0)

        def pair_body(p, _):
            c0 = 2 * p
            start_in(c0 + 1, 1)

            pl.when(p > 0)(lambda: wait_out(0))
            wait_in(0)
            compute(0)
            start_out(c0, 0)

            pl.when(p < ch // 2 - 1)(lambda: start_in(c0 + 2, 0))

            pl.when(p > 0)(lambda: wait_out(1))
            wait_in(1)
            compute(1)
            start_out(c0 + 1, 1)
            return 0

        lax.fori_loop(0, ch // 2, pair_body, 0)
        wait_out(0)
        wait_out(1)

    return k


_sc_kernel = _make_sc_kernel(1024 * 200)


@jax.jit
def kernel(x, table, gamma, beta):
    b, s = x.shape
    idx = x.reshape(NW, b * s // (NW * GB), GB).astype(jnp.int32)
    out = _sc_kernel(table, idx, gamma, beta)
    return out.reshape(b, s, D)
